# baseline (speedup 1.0000x reference)
import functools

import jax
import jax.numpy as jnp
from jax import lax
from jax.experimental import pallas as pl
from jax.experimental.pallas import tpu as pltpu
from jax.experimental.pallas import tpu_sc as plsc

N = 10000
E = 320000
D = 128
H = 128
C = 40

NC = 2
NS = 16
NW = NC * NS
EPT = E // NW
CH = 80
NCHP = 128
EPT2 = NCHP * CH
EPAD = NW * EPT2
NPAD = N + 8
NBUF = 3
LOOK = 2
DHALF = NCHP // 2
OUTER = (NCHP + NBUF - 1) // NBUF

_MESH = plsc.VectorSubcoreMesh(core_axis_name="c", subcore_axis_name="s")


@functools.partial(
    pl.kernel,
    out_type=jax.ShapeDtypeStruct((NC, N), jnp.float32),
    mesh=_MESH,
    compiler_params=pltpu.CompilerParams(needs_layout_passes=False),
    scratch_types=[
        pltpu.VMEM((EPT,), jnp.int32),
        pltpu.VMEM((N,), jnp.float32),
        pltpu.VMEM((N,), jnp.float32),
        pltpu.VMEM_SHARED((NS, N), jnp.float32),
    ],
)
def _sc_deg(dst_hbm, pdeg_out, dst_v, pdeg_v, red_v, stage_sh):
    c = lax.axis_index("c")
    s = lax.axis_index("s")
    tid = c * NS + s
    pltpu.sync_copy(dst_hbm.at[pl.ds(tid * EPT, EPT)], dst_v)
    ones = jnp.full((16,), 1.0, dtype=jnp.float32)
    zeros = jnp.zeros((16,), dtype=jnp.float32)

    def zero_body(i, _):
        pdeg_v[pl.ds(i * 16, 16)] = zeros
        return 0

    lax.fori_loop(0, N // 16, zero_body, 0)

    def count_body(i, _):
        idx = dst_v[pl.ds(i * 16, 16)]
        plsc.addupdate_scatter(pdeg_v, [idx], ones)
        return 0

    lax.fori_loop(0, EPT // 16, count_body, 0)
    pltpu.sync_copy(pdeg_v, stage_sh.at[s])
    plsc.subcore_barrier()

    for step in (8, 4, 2, 1):
        @pl.when(s < step)
        def _():
            pltpu.sync_copy(stage_sh.at[s + step], red_v)

            def red_body(i, _):
                sl = pl.ds(i * 16, 16)
                pdeg_v[sl] = pdeg_v[sl] + red_v[sl]
                return 0

            lax.fori_loop(0, N // 16, red_body, 0)
            if step > 1:
                pltpu.sync_copy(pdeg_v, stage_sh.at[s])
        plsc.subcore_barrier()

    @pl.when(s == 0)
    def _():
        pltpu.sync_copy(pdeg_v, pdeg_out.at[c])


@functools.partial(
    pl.kernel,
    out_type=jax.ShapeDtypeStruct((NC, N, H), jnp.float32),
    mesh=_MESH,
    compiler_params=pltpu.CompilerParams(needs_layout_passes=False),
    scratch_types=[
        pltpu.VMEM((EPT2,), jnp.int32),
        pltpu.VMEM((DHALF, CH), jnp.int32),
        pltpu.VMEM((NBUF, CH, H), jnp.float32),
        pltpu.VMEM_SHARED((NPAD, H), jnp.float32),
        pltpu.SemaphoreType.DMA,
        pltpu.SemaphoreType.DMA,
        pltpu.SemaphoreType.DMA,
        pltpu.SemaphoreType.DMA,
        pltpu.SemaphoreType.DMA,
        pltpu.SemaphoreType.DMA,
    ],
)
def _sc_agg(hp_hbm, src_hbm, dst_hbm, zeros_hbm, acc_out,
            src_v, dst_v, rows_v, acc_sh,
            g0, g1, g2, s0, s1, s2):
    gsem = (g0, g1, g2)
    ssem = (s0, s1, s2)
    c = lax.axis_index("c")
    s = lax.axis_index("s")
    tid = c * NS + s
    pltpu.sync_copy(src_hbm.at[pl.ds(tid * EPT2, EPT2)], src_v)
    pltpu.sync_copy(dst_hbm.at[tid, pl.ds(0, DHALF)], dst_v)
    nch = jnp.minimum((E - tid * EPT2) // CH, NCHP)

    for b in range(LOOK):
        pltpu.async_copy(hp_hbm.at[src_v.at[pl.ds(b * CH, CH)]],
                         rows_v.at[b], gsem[b])

    @pl.when(s < NS - 1)
    def _():
        pltpu.sync_copy(zeros_hbm.at[pl.ds(s * 624, 624)],
                        acc_sh.at[pl.ds(s * 624, 624)])

    @pl.when(s == NS - 1)
    def _():
        pltpu.sync_copy(zeros_hbm.at[pl.ds(9360, 640)],
                        acc_sh.at[pl.ds(9360, 640)])
    plsc.subcore_barrier()

    def outer(i, _):
        g = i * NBUF
        for b in range(NBUF):
            j = g + b
            k = j + LOOK
            kb = (b + LOOK) % NBUF

            @pl.when(k < nch)
            def _():
                @pl.when(k >= NBUF)
                def _():
                    pltpu.make_async_copy(
                        hp_hbm.at[pl.ds(0, CH)], rows_v.at[kb],
                        ssem[kb]).wait()
                pltpu.async_copy(hp_hbm.at[src_v.at[pl.ds(k * CH, CH)]],
                                 rows_v.at[kb], gsem[kb])

            @pl.when((j == DHALF) & (DHALF < nch))
            def _():
                pltpu.sync_copy(dst_hbm.at[tid, pl.ds(DHALF, DHALF)], dst_v)

            @pl.when(j < nch)
            def _():
                pltpu.make_async_copy(
                    hp_hbm.at[pl.ds(0, CH)], rows_v.at[b], gsem[b]).wait()
                jr = jnp.where(j >= DHALF, j - DHALF, j)
                pltpu.async_copy(rows_v.at[b], acc_sh.at[dst_v.at[jr]],
                                 ssem[b], add=True)
        return 0

    lax.fori_loop(0, OUTER, outer, 0)
    for b in range(NBUF):
        pltpu.make_async_copy(
            hp_hbm.at[pl.ds(0, CH)], rows_v.at[b], ssem[b]).wait()
    plsc.subcore_barrier()

    @pl.when(s < NS - 1)
    def _():
        pltpu.sync_copy(acc_sh.at[pl.ds(s * 624, 624)],
                        acc_out.at[c, pl.ds(s * 624, 624)])

    @pl.when(s == NS - 1)
    def _():
        pltpu.sync_copy(acc_sh.at[pl.ds(9360, 640)],
                        acc_out.at[c, pl.ds(9360, 640)])


RB = 1000
GRID = N // RB


def _dis_block(pdeg_ref):
    deg = 1.0 + pdeg_ref[:, 0] + pdeg_ref[:, 1]
    return lax.rsqrt(deg)


def _tc_a1_body(x_ref, w_ref, pdeg_ref, o_ref, dis_ref, z_ref):
    deg = 1.0 + pdeg_ref[:, 0] + pdeg_ref[:, 1]
    dis = lax.rsqrt(deg)
    dis_ref[...] = dis[:, None]
    h = jnp.dot(x_ref[...], w_ref[...], preferred_element_type=jnp.float32)
    o_ref[...] = h * dis[:, None]
    z_ref[...] = jnp.zeros((RB, H), jnp.float32)


def _tc_a1(x, w_in, pdeg):
    return pl.pallas_call(
        _tc_a1_body,
        grid=(GRID,),
        in_specs=[
            pl.BlockSpec((RB, D), lambda i: (i, 0)),
            pl.BlockSpec((D, H), lambda i: (0, 0)),
            pl.BlockSpec((RB, NC), lambda i: (i, 0)),
        ],
        out_specs=[
            pl.BlockSpec((RB, H), lambda i: (i, 0)),
            pl.BlockSpec((RB, 1), lambda i: (i, 0)),
            pl.BlockSpec((RB, H), lambda i: (i, 0)),
        ],
        out_shape=[
            jax.ShapeDtypeStruct((N, H), jnp.float32),
            jax.ShapeDtypeStruct((N, 1), jnp.float32),
            jax.ShapeDtypeStruct((N, H), jnp.float32),
        ],
    )(x, w_in, pdeg)


def _tc_a2_body(acc_ref, hp_ref, dis_ref, b_ref, w_ref, o_ref):
    dis = dis_ref[...]
    agg = acc_ref[0] + acc_ref[1] + hp_ref[...]
    h1 = agg * dis + b_ref[...]
    h2 = jnp.dot(h1, w_ref[...], preferred_element_type=jnp.float32)
    o_ref[...] = h2 * dis


def _tc_a2(acc, hp, dis, b, w):
    return pl.pallas_call(
        _tc_a2_body,
        grid=(GRID,),
        in_specs=[
            pl.BlockSpec((NC, RB, H), lambda i: (0, i, 0)),
            pl.BlockSpec((RB, H), lambda i: (i, 0)),
            pl.BlockSpec((RB, 1), lambda i: (i, 0)),
            pl.BlockSpec((1, H), lambda i: (0, 0)),
            pl.BlockSpec((H, H), lambda i: (0, 0)),
        ],
        out_specs=pl.BlockSpec((RB, H), lambda i: (i, 0)),
        out_shape=jax.ShapeDtypeStruct((N, H), jnp.float32),
    )(acc, hp, dis, b, w)


def _tc_a3_body(acc_ref, hp_ref, dis_ref, b1_ref, wout_ref, bout_ref, o_ref):
    dis = dis_ref[...]
    agg = acc_ref[0] + acc_ref[1] + hp_ref[...]
    h2 = jnp.maximum(agg * dis + b1_ref[...], 0.0)
    logits = jnp.dot(h2, wout_ref[...], preferred_element_type=jnp.float32)
    logits = logits + bout_ref[...]
    m = jnp.max(logits, axis=1, keepdims=True)
    z = logits - m
    lse = jnp.log(jnp.sum(jnp.exp(z), axis=1, keepdims=True))
    o_ref[...] = z - lse


def _tc_a3(acc, hp, dis, b1, w_out, b_out):
    return pl.pallas_call(
        _tc_a3_body,
        grid=(GRID,),
        in_specs=[
            pl.BlockSpec((NC, RB, H), lambda i: (0, i, 0)),
            pl.BlockSpec((RB, H), lambda i: (i, 0)),
            pl.BlockSpec((RB, 1), lambda i: (i, 0)),
            pl.BlockSpec((1, H), lambda i: (0, 0)),
            pl.BlockSpec((H, C), lambda i: (0, 0)),
            pl.BlockSpec((1, C), lambda i: (0, 0)),
        ],
        out_specs=pl.BlockSpec((RB, C), lambda i: (i, 0)),
        out_shape=jax.ShapeDtypeStruct((N, C), jnp.float32),
    )(acc, hp, dis, b1, w_out, b_out)


def kernel(x, edge_index, W_in, b_in, W1, b1, W_out, b_out):
    src = edge_index[0]
    dst = edge_index[1]
    pad = EPAD - E
    src_r = jnp.concatenate([src, jnp.zeros((pad,), jnp.int32)])
    dst_r = jnp.concatenate(
        [dst, jnp.full((pad,), N, jnp.int32)]).reshape(NW, NCHP, CH)
    pdeg = _sc_deg(dst).T
    h1p, dis, zeros_big = _tc_a1(x, W_in, pdeg)
    acc1 = _sc_agg(h1p, src_r, dst_r, zeros_big)
    h2p = _tc_a2(acc1, h1p, dis, b_in.reshape(1, H), W1)
    acc2 = _sc_agg(h2p, src_r, dst_r, zeros_big)
    out = _tc_a3(acc2, h2p, dis, b1.reshape(1, H), W_out.reshape(H, C),
                 b_out.reshape(1, C))
    return out


# speedup vs baseline: 1.0063x; 1.0063x over previous
import functools

import jax
import jax.numpy as jnp
from jax import lax
from jax.experimental import pallas as pl
from jax.experimental.pallas import tpu as pltpu
from jax.experimental.pallas import tpu_sc as plsc

N = 10000
E = 320000
D = 128
H = 128
C = 40

NC = 2
NS = 16
NW = NC * NS
EPT = E // NW
CH = 80
NCHP = 128
EPT2 = NCHP * CH
EPAD = NW * EPT2
NPAD = N + 8
NBUF = 3
LOOK = 2
DHALF = NCHP // 2
OUTER = (NCHP + NBUF - 1) // NBUF

_MESH = plsc.VectorSubcoreMesh(core_axis_name="c", subcore_axis_name="s")


@functools.partial(
    pl.kernel,
    out_type=jax.ShapeDtypeStruct((NC, N), jnp.float32),
    mesh=_MESH,
    compiler_params=pltpu.CompilerParams(needs_layout_passes=False),
    scratch_types=[
        pltpu.VMEM((EPT,), jnp.int32),
        pltpu.VMEM((N,), jnp.float32),
        pltpu.VMEM((N,), jnp.float32),
        pltpu.VMEM_SHARED((NS, N), jnp.float32),
    ],
)
def _sc_deg(dst_hbm, pdeg_out, dst_v, pdeg_v, red_v, stage_sh):
    c = lax.axis_index("c")
    s = lax.axis_index("s")
    tid = c * NS + s
    pltpu.sync_copy(dst_hbm.at[pl.ds(tid * EPT, EPT)], dst_v)
    ones = jnp.full((16,), 1.0, dtype=jnp.float32)
    zeros = jnp.zeros((16,), dtype=jnp.float32)

    def zero_body(i, _):
        pdeg_v[pl.ds(i * 16, 16)] = zeros
        return 0

    lax.fori_loop(0, N // 16, zero_body, 0)

    def count_body(i, _):
        idx = dst_v[pl.ds(i * 16, 16)]
        plsc.addupdate_scatter(pdeg_v, [idx], ones)
        return 0

    lax.fori_loop(0, EPT // 16, count_body, 0)
    pltpu.sync_copy(pdeg_v, stage_sh.at[s])
    plsc.subcore_barrier()

    for step in (8, 4, 2, 1):
        @pl.when(s < step)
        def _():
            pltpu.sync_copy(stage_sh.at[s + step], red_v)

            def red_body(i, _):
                sl = pl.ds(i * 16, 16)
                pdeg_v[sl] = pdeg_v[sl] + red_v[sl]
                return 0

            lax.fori_loop(0, N // 16, red_body, 0)
            if step > 1:
                pltpu.sync_copy(pdeg_v, stage_sh.at[s])
        plsc.subcore_barrier()

    @pl.when(s == 0)
    def _():
        pltpu.sync_copy(pdeg_v, pdeg_out.at[c])


@functools.partial(
    pl.kernel,
    out_type=jax.ShapeDtypeStruct((NC, N, H), jnp.float32),
    mesh=_MESH,
    compiler_params=pltpu.CompilerParams(needs_layout_passes=False),
    scratch_types=[
        pltpu.VMEM((EPT2,), jnp.int32),
        pltpu.VMEM((DHALF, CH), jnp.int32),
        pltpu.VMEM((NBUF, CH, H), jnp.float32),
        pltpu.VMEM_SHARED((NPAD, H), jnp.float32),
        pltpu.SemaphoreType.DMA,
        pltpu.SemaphoreType.DMA,
        pltpu.SemaphoreType.DMA,
        pltpu.SemaphoreType.DMA,
        pltpu.SemaphoreType.DMA,
        pltpu.SemaphoreType.DMA,
    ],
)
def _sc_agg(hp_hbm, src_hbm, dst_hbm, zeros_hbm, acc_out,
            src_v, dst_v, rows_v, acc_sh,
            g0, g1, g2, s0, s1, s2):
    gsem = (g0, g1, g2)
    ssem = (s0, s1, s2)
    c = lax.axis_index("c")
    s = lax.axis_index("s")
    tid = c * NS + s
    pltpu.sync_copy(src_hbm.at[pl.ds(tid * EPT2, EPT2)], src_v)
    pltpu.sync_copy(dst_hbm.at[tid, pl.ds(0, DHALF)], dst_v)
    nch = jnp.minimum((E - tid * EPT2) // CH, NCHP)

    for b in range(LOOK):
        pltpu.async_copy(hp_hbm.at[src_v.at[pl.ds(b * CH, CH)]],
                         rows_v.at[b], gsem[b])

    @pl.when(s < NS - 1)
    def _():
        pltpu.sync_copy(zeros_hbm.at[pl.ds(s * 624, 624)],
                        acc_sh.at[pl.ds(s * 624, 624)])

    @pl.when(s == NS - 1)
    def _():
        pltpu.sync_copy(zeros_hbm.at[pl.ds(9360, 640)],
                        acc_sh.at[pl.ds(9360, 640)])
    plsc.subcore_barrier()

    def outer(i, _):
        g = i * NBUF
        for b in range(NBUF):
            j = g + b
            k = j + LOOK
            kb = (b + LOOK) % NBUF

            @pl.when(k < nch)
            def _():
                @pl.when(k >= NBUF)
                def _():
                    pltpu.make_async_copy(
                        hp_hbm.at[pl.ds(0, CH)], rows_v.at[kb],
                        ssem[kb]).wait()
                pltpu.async_copy(hp_hbm.at[src_v.at[pl.ds(k * CH, CH)]],
                                 rows_v.at[kb], gsem[kb])

            @pl.when((j == DHALF) & (DHALF < nch))
            def _():
                pltpu.sync_copy(dst_hbm.at[tid, pl.ds(DHALF, DHALF)], dst_v)

            @pl.when(j < nch)
            def _():
                pltpu.make_async_copy(
                    hp_hbm.at[pl.ds(0, CH)], rows_v.at[b], gsem[b]).wait()
                jr = jnp.where(j >= DHALF, j - DHALF, j)
                pltpu.async_copy(rows_v.at[b], acc_sh.at[dst_v.at[jr]],
                                 ssem[b], add=True)
        return 0

    lax.fori_loop(0, OUTER, outer, 0)
    for b in range(NBUF):
        pltpu.make_async_copy(
            hp_hbm.at[pl.ds(0, CH)], rows_v.at[b], ssem[b]).wait()
    plsc.subcore_barrier()

    @pl.when(s < NS - 1)
    def _():
        pltpu.sync_copy(acc_sh.at[pl.ds(s * 624, 624)],
                        acc_out.at[c, pl.ds(s * 624, 624)])

    @pl.when(s == NS - 1)
    def _():
        pltpu.sync_copy(acc_sh.at[pl.ds(9360, 640)],
                        acc_out.at[c, pl.ds(9360, 640)])


RB = 1000
GRID = N // RB


def _dis_block(pdeg_ref):
    deg = 1.0 + pdeg_ref[:, 0] + pdeg_ref[:, 1]
    return lax.rsqrt(deg)


def _tc_m1_body(x_ref, w_ref, o_ref, z_ref):
    o_ref[...] = jnp.dot(x_ref[...], w_ref[...],
                         preferred_element_type=jnp.float32)
    z_ref[...] = jnp.zeros((RB, H), jnp.float32)


def _tc_m1(x, w_in):
    return pl.pallas_call(
        _tc_m1_body,
        grid=(GRID,),
        in_specs=[
            pl.BlockSpec((RB, D), lambda i: (i, 0)),
            pl.BlockSpec((D, H), lambda i: (0, 0)),
        ],
        out_specs=[
            pl.BlockSpec((RB, H), lambda i: (i, 0)),
            pl.BlockSpec((RB, H), lambda i: (i, 0)),
        ],
        out_shape=[
            jax.ShapeDtypeStruct((N, H), jnp.float32),
            jax.ShapeDtypeStruct((N, H), jnp.float32),
        ],
    )(x, w_in)


def _tc_s1_body(h_ref, pdeg_ref, o_ref, dis_ref):
    deg = 1.0 + pdeg_ref[:, 0] + pdeg_ref[:, 1]
    dis = lax.rsqrt(deg)
    dis_ref[...] = dis[:, None]
    o_ref[...] = h_ref[...] * dis[:, None]


def _tc_s1(h, pdeg):
    return pl.pallas_call(
        _tc_s1_body,
        grid=(GRID,),
        in_specs=[
            pl.BlockSpec((RB, H), lambda i: (i, 0)),
            pl.BlockSpec((RB, NC), lambda i: (i, 0)),
        ],
        out_specs=[
            pl.BlockSpec((RB, H), lambda i: (i, 0)),
            pl.BlockSpec((RB, 1), lambda i: (i, 0)),
        ],
        out_shape=[
            jax.ShapeDtypeStruct((N, H), jnp.float32),
            jax.ShapeDtypeStruct((N, 1), jnp.float32),
        ],
    )(h, pdeg)


def _tc_a2_body(acc_ref, hp_ref, dis_ref, b_ref, w_ref, o_ref):
    dis = dis_ref[...]
    agg = acc_ref[0] + acc_ref[1] + hp_ref[...]
    h1 = agg * dis + b_ref[...]
    h2 = jnp.dot(h1, w_ref[...], preferred_element_type=jnp.float32)
    o_ref[...] = h2 * dis


def _tc_a2(acc, hp, dis, b, w):
    return pl.pallas_call(
        _tc_a2_body,
        grid=(GRID,),
        in_specs=[
            pl.BlockSpec((NC, RB, H), lambda i: (0, i, 0)),
            pl.BlockSpec((RB, H), lambda i: (i, 0)),
            pl.BlockSpec((RB, 1), lambda i: (i, 0)),
            pl.BlockSpec((1, H), lambda i: (0, 0)),
            pl.BlockSpec((H, H), lambda i: (0, 0)),
        ],
        out_specs=pl.BlockSpec((RB, H), lambda i: (i, 0)),
        out_shape=jax.ShapeDtypeStruct((N, H), jnp.float32),
    )(acc, hp, dis, b, w)


def _tc_a3_body(acc_ref, hp_ref, dis_ref, b1_ref, wout_ref, bout_ref, o_ref):
    dis = dis_ref[...]
    agg = acc_ref[0] + acc_ref[1] + hp_ref[...]
    h2 = jnp.maximum(agg * dis + b1_ref[...], 0.0)
    logits = jnp.dot(h2, wout_ref[...], preferred_element_type=jnp.float32)
    logits = logits + bout_ref[...]
    m = jnp.max(logits, axis=1, keepdims=True)
    z = logits - m
    lse = jnp.log(jnp.sum(jnp.exp(z), axis=1, keepdims=True))
    o_ref[...] = z - lse


def _tc_a3(acc, hp, dis, b1, w_out, b_out):
    return pl.pallas_call(
        _tc_a3_body,
        grid=(GRID,),
        in_specs=[
            pl.BlockSpec((NC, RB, H), lambda i: (0, i, 0)),
            pl.BlockSpec((RB, H), lambda i: (i, 0)),
            pl.BlockSpec((RB, 1), lambda i: (i, 0)),
            pl.BlockSpec((1, H), lambda i: (0, 0)),
            pl.BlockSpec((H, C), lambda i: (0, 0)),
            pl.BlockSpec((1, C), lambda i: (0, 0)),
        ],
        out_specs=pl.BlockSpec((RB, C), lambda i: (i, 0)),
        out_shape=jax.ShapeDtypeStruct((N, C), jnp.float32),
    )(acc, hp, dis, b1, w_out, b_out)


def kernel(x, edge_index, W_in, b_in, W1, b1, W_out, b_out):
    src = edge_index[0]
    dst = edge_index[1]
    pad = EPAD - E
    src_r = jnp.concatenate([src, jnp.zeros((pad,), jnp.int32)])
    dst_r = jnp.concatenate(
        [dst, jnp.full((pad,), N, jnp.int32)]).reshape(NW, NCHP, CH)
    pdeg = _sc_deg(dst).T
    h0, zeros_big = _tc_m1(x, W_in)
    h1p, dis = _tc_s1(h0, pdeg)
    acc1 = _sc_agg(h1p, src_r, dst_r, zeros_big)
    h2p = _tc_a2(acc1, h1p, dis, b_in.reshape(1, H), W1)
    acc2 = _sc_agg(h2p, src_r, dst_r, zeros_big)
    out = _tc_a3(acc2, h2p, dis, b1.reshape(1, H), W_out.reshape(H, C),
                 b_out.reshape(1, C))
    return out


# speedup vs baseline: 1.0345x; 1.0280x over previous
import functools

import jax
import jax.numpy as jnp
from jax import lax
from jax.experimental import pallas as pl
from jax.experimental.pallas import tpu as pltpu
from jax.experimental.pallas import tpu_sc as plsc

N = 10000
E = 320000
D = 128
H = 128
C = 40

NC = 2
NS = 16
NW = NC * NS
EPT = E // NW
CH = 80
NCHP = 128
EPT2 = NCHP * CH
EPAD = NW * EPT2
NPAD = N + 8
NDEG = 10240
CSL = NDEG // NS
NBUF = 3
LOOK = 2
DHALF = NCHP // 2
OUTER = (NCHP + NBUF - 1) // NBUF

_MESH = plsc.VectorSubcoreMesh(core_axis_name="c", subcore_axis_name="s")


@functools.partial(
    pl.kernel,
    out_type=jax.ShapeDtypeStruct((NC, NDEG), jnp.float32),
    mesh=_MESH,
    compiler_params=pltpu.CompilerParams(needs_layout_passes=False),
    scratch_types=[
        pltpu.VMEM((EPT,), jnp.int32),
        pltpu.VMEM((NDEG,), jnp.float32),
        pltpu.VMEM((NS, CSL), jnp.float32),
        pltpu.VMEM((CSL,), jnp.float32),
        pltpu.VMEM_SHARED((NS, NDEG), jnp.float32),
    ],
)
def _sc_deg(dst_hbm, pdeg_out, dst_v, pdeg_v, red_v, out_v, stage_sh):
    c = lax.axis_index("c")
    s = lax.axis_index("s")
    tid = c * NS + s
    pltpu.sync_copy(dst_hbm.at[pl.ds(tid * EPT, EPT)], dst_v)
    ones = jnp.full((16,), 1.0, dtype=jnp.float32)
    zeros = jnp.zeros((16,), dtype=jnp.float32)

    def zero_body(i, _):
        pdeg_v[pl.ds(i * 16, 16)] = zeros
        return 0

    lax.fori_loop(0, NDEG // 16, zero_body, 0)

    def count_body(i, _):
        idx = dst_v[pl.ds(i * 16, 16)]
        plsc.addupdate_scatter(pdeg_v, [idx], ones)
        return 0

    lax.fori_loop(0, EPT // 16, count_body, 0)
    pltpu.sync_copy(pdeg_v, stage_sh.at[s])
    plsc.subcore_barrier()

    pltpu.sync_copy(stage_sh.at[:, pl.ds(s * CSL, CSL)], red_v)

    def red_body(i, _):
        sl = pl.ds(i * 16, 16)
        acc = red_v[0, sl]
        for r in range(1, NS):
            acc = acc + red_v[r, sl]
        out_v[sl] = acc
        return 0

    lax.fori_loop(0, CSL // 16, red_body, 0)
    pltpu.sync_copy(out_v, pdeg_out.at[c, pl.ds(s * CSL, CSL)])


@functools.partial(
    pl.kernel,
    out_type=jax.ShapeDtypeStruct((NC, N, H), jnp.float32),
    mesh=_MESH,
    compiler_params=pltpu.CompilerParams(needs_layout_passes=False),
    scratch_types=[
        pltpu.VMEM((EPT2,), jnp.int32),
        pltpu.VMEM((DHALF, CH), jnp.int32),
        pltpu.VMEM((NBUF, CH, H), jnp.float32),
        pltpu.VMEM_SHARED((NPAD, H), jnp.float32),
        pltpu.SemaphoreType.DMA,
        pltpu.SemaphoreType.DMA,
        pltpu.SemaphoreType.DMA,
        pltpu.SemaphoreType.DMA,
        pltpu.SemaphoreType.DMA,
        pltpu.SemaphoreType.DMA,
    ],
)
def _sc_agg(hp_hbm, src_hbm, dst_hbm, zeros_hbm, acc_out,
            src_v, dst_v, rows_v, acc_sh,
            g0, g1, g2, s0, s1, s2):
    gsem = (g0, g1, g2)
    ssem = (s0, s1, s2)
    c = lax.axis_index("c")
    s = lax.axis_index("s")
    tid = c * NS + s
    pltpu.sync_copy(src_hbm.at[pl.ds(tid * EPT2, EPT2)], src_v)
    pltpu.sync_copy(dst_hbm.at[tid, pl.ds(0, DHALF)], dst_v)
    nch = jnp.minimum((E - tid * EPT2) // CH, NCHP)

    for b in range(LOOK):
        pltpu.async_copy(hp_hbm.at[src_v.at[pl.ds(b * CH, CH)]],
                         rows_v.at[b], gsem[b])

    @pl.when(s < NS - 1)
    def _():
        pltpu.sync_copy(zeros_hbm.at[pl.ds(s * 624, 624)],
                        acc_sh.at[pl.ds(s * 624, 624)])

    @pl.when(s == NS - 1)
    def _():
        pltpu.sync_copy(zeros_hbm.at[pl.ds(9360, 640)],
                        acc_sh.at[pl.ds(9360, 640)])
    plsc.subcore_barrier()

    def outer(i, _):
        g = i * NBUF
        for b in range(NBUF):
            j = g + b
            k = j + LOOK
            kb = (b + LOOK) % NBUF

            @pl.when(k < nch)
            def _():
                @pl.when(k >= NBUF)
                def _():
                    pltpu.make_async_copy(
                        hp_hbm.at[pl.ds(0, CH)], rows_v.at[kb],
                        ssem[kb]).wait()
                pltpu.async_copy(hp_hbm.at[src_v.at[pl.ds(k * CH, CH)]],
                                 rows_v.at[kb], gsem[kb])

            @pl.when((j == DHALF) & (DHALF < nch))
            def _():
                pltpu.sync_copy(dst_hbm.at[tid, pl.ds(DHALF, DHALF)], dst_v)

            @pl.when(j < nch)
            def _():
                pltpu.make_async_copy(
                    hp_hbm.at[pl.ds(0, CH)], rows_v.at[b], gsem[b]).wait()
                jr = jnp.where(j >= DHALF, j - DHALF, j)
                pltpu.async_copy(rows_v.at[b], acc_sh.at[dst_v.at[jr]],
                                 ssem[b], add=True)
        return 0

    lax.fori_loop(0, OUTER, outer, 0)
    for b in range(NBUF):
        pltpu.make_async_copy(
            hp_hbm.at[pl.ds(0, CH)], rows_v.at[b], ssem[b]).wait()
    plsc.subcore_barrier()

    @pl.when(s < NS - 1)
    def _():
        pltpu.sync_copy(acc_sh.at[pl.ds(s * 624, 624)],
                        acc_out.at[c, pl.ds(s * 624, 624)])

    @pl.when(s == NS - 1)
    def _():
        pltpu.sync_copy(acc_sh.at[pl.ds(9360, 640)],
                        acc_out.at[c, pl.ds(9360, 640)])


RB = 1000
GRID = N // RB


def _dis_block(pdeg_ref):
    deg = 1.0 + pdeg_ref[:, 0] + pdeg_ref[:, 1]
    return lax.rsqrt(deg)


def _tc_m1_body(x_ref, w_ref, o_ref, z_ref):
    o_ref[...] = jnp.dot(x_ref[...], w_ref[...],
                         preferred_element_type=jnp.float32)
    z_ref[...] = jnp.zeros((RB, H), jnp.float32)


def _tc_m1(x, w_in):
    return pl.pallas_call(
        _tc_m1_body,
        grid=(GRID,),
        in_specs=[
            pl.BlockSpec((RB, D), lambda i: (i, 0)),
            pl.BlockSpec((D, H), lambda i: (0, 0)),
        ],
        out_specs=[
            pl.BlockSpec((RB, H), lambda i: (i, 0)),
            pl.BlockSpec((RB, H), lambda i: (i, 0)),
        ],
        out_shape=[
            jax.ShapeDtypeStruct((N, H), jnp.float32),
            jax.ShapeDtypeStruct((N, H), jnp.float32),
        ],
    )(x, w_in)


def _tc_s1_body(h_ref, pdeg_ref, o_ref, dis_ref):
    deg = 1.0 + pdeg_ref[:, 0] + pdeg_ref[:, 1]
    dis = lax.rsqrt(deg)
    dis_ref[...] = dis[:, None]
    o_ref[...] = h_ref[...] * dis[:, None]


def _tc_s1(h, pdeg):
    return pl.pallas_call(
        _tc_s1_body,
        grid=(GRID,),
        in_specs=[
            pl.BlockSpec((RB, H), lambda i: (i, 0)),
            pl.BlockSpec((RB, NC), lambda i: (i, 0)),
        ],
        out_specs=[
            pl.BlockSpec((RB, H), lambda i: (i, 0)),
            pl.BlockSpec((RB, 1), lambda i: (i, 0)),
        ],
        out_shape=[
            jax.ShapeDtypeStruct((N, H), jnp.float32),
            jax.ShapeDtypeStruct((N, 1), jnp.float32),
        ],
    )(h, pdeg)


def _tc_a2_body(acc_ref, hp_ref, dis_ref, b_ref, w_ref, o_ref):
    dis = dis_ref[...]
    agg = acc_ref[0] + acc_ref[1] + hp_ref[...]
    h1 = agg * dis + b_ref[...]
    h2 = jnp.dot(h1, w_ref[...], preferred_element_type=jnp.float32)
    o_ref[...] = h2 * dis


def _tc_a2(acc, hp, dis, b, w):
    return pl.pallas_call(
        _tc_a2_body,
        grid=(GRID,),
        in_specs=[
            pl.BlockSpec((NC, RB, H), lambda i: (0, i, 0)),
            pl.BlockSpec((RB, H), lambda i: (i, 0)),
            pl.BlockSpec((RB, 1), lambda i: (i, 0)),
            pl.BlockSpec((1, H), lambda i: (0, 0)),
            pl.BlockSpec((H, H), lambda i: (0, 0)),
        ],
        out_specs=pl.BlockSpec((RB, H), lambda i: (i, 0)),
        out_shape=jax.ShapeDtypeStruct((N, H), jnp.float32),
    )(acc, hp, dis, b, w)


def _tc_a3_body(acc_ref, hp_ref, dis_ref, b1_ref, wout_ref, bout_ref, o_ref):
    dis = dis_ref[...]
    agg = acc_ref[0] + acc_ref[1] + hp_ref[...]
    h2 = jnp.maximum(agg * dis + b1_ref[...], 0.0)
    logits = jnp.dot(h2, wout_ref[...], preferred_element_type=jnp.float32)
    logits = logits + bout_ref[...]
    m = jnp.max(logits, axis=1, keepdims=True)
    z = logits - m
    lse = jnp.log(jnp.sum(jnp.exp(z), axis=1, keepdims=True))
    o_ref[...] = z - lse


def _tc_a3(acc, hp, dis, b1, w_out, b_out):
    return pl.pallas_call(
        _tc_a3_body,
        grid=(GRID,),
        in_specs=[
            pl.BlockSpec((NC, RB, H), lambda i: (0, i, 0)),
            pl.BlockSpec((RB, H), lambda i: (i, 0)),
            pl.BlockSpec((RB, 1), lambda i: (i, 0)),
            pl.BlockSpec((1, H), lambda i: (0, 0)),
            pl.BlockSpec((H, C), lambda i: (0, 0)),
            pl.BlockSpec((1, C), lambda i: (0, 0)),
        ],
        out_specs=pl.BlockSpec((RB, C), lambda i: (i, 0)),
        out_shape=jax.ShapeDtypeStruct((N, C), jnp.float32),
    )(acc, hp, dis, b1, w_out, b_out)


def kernel(x, edge_index, W_in, b_in, W1, b1, W_out, b_out):
    src = edge_index[0]
    dst = edge_index[1]
    pad = EPAD - E
    src_r = jnp.concatenate([src, jnp.zeros((pad,), jnp.int32)])
    dst_r = jnp.concatenate(
        [dst, jnp.full((pad,), N, jnp.int32)]).reshape(NW, NCHP, CH)
    pdeg = _sc_deg(dst).T[:N]
    h0, zeros_big = _tc_m1(x, W_in)
    h1p, dis = _tc_s1(h0, pdeg)
    acc1 = _sc_agg(h1p, src_r, dst_r, zeros_big)
    h2p = _tc_a2(acc1, h1p, dis, b_in.reshape(1, H), W1)
    acc2 = _sc_agg(h2p, src_r, dst_r, zeros_big)
    out = _tc_a3(acc2, h2p, dis, b1.reshape(1, H), W_out.reshape(H, C),
                 b_out.reshape(1, C))
    return out
